# baseline (speedup 1.0000x reference)
import jax
import jax.numpy as jnp
from jax import lax
from jax.experimental import pallas as pl
from jax.experimental.pallas import tpu as pltpu
from jax.experimental.pallas import tpu_sc as plsc

N = 10000
E = 160000
D = 128
DA = 144
D_EDGE = 16
NEG = 0.2

NC = 2
NS = 16
NW = NC * NS
C = 40
NCHUNK = E // C
NPW = NCHUNK // NW
NBLK = N // C


def _fused_mm_body(x_ref, wl_ref, bl_ref, wr_ref, br_ref, ea_ref, we_ref,
                   xl_ref, xr_ref, e_ref):
    x = x_ref[...]
    xl_ref[...] = jnp.dot(x, wl_ref[...], preferred_element_type=jnp.float32) + bl_ref[...]
    xr_ref[...] = jnp.dot(x, wr_ref[...], preferred_element_type=jnp.float32) + br_ref[...]
    e_ref[...] = jnp.dot(ea_ref[...], we_ref[...], preferred_element_type=jnp.float32)


def _combine_body(u0_ref, u1_ref, dt_ref, b_ref, o_ref):
    den = jnp.sum(dt_ref[...], axis=1, keepdims=True)
    o_ref[...] = (u0_ref[0] + u1_ref[0]) / (den + 1e-16) + b_ref[...]


def _tc_matmuls(x, W_l, b_l, W_r, b_r, edge_attr, W_e):
    BN = 1000
    BE = E // 8 // 10
    G = 10
    ea2 = edge_attr.reshape(E // 8, 8 * D_EDGE)
    w_bd = jnp.kron(jnp.eye(8, dtype=jnp.float32), W_e)
    xl, xr, e = pl.pallas_call(
        _fused_mm_body,
        grid=(G,),
        in_specs=[
            pl.BlockSpec((BN, D), lambda i: (i, 0)),
            pl.BlockSpec((D, D), lambda i: (0, 0)),
            pl.BlockSpec((1, D), lambda i: (0, 0)),
            pl.BlockSpec((D, D), lambda i: (0, 0)),
            pl.BlockSpec((1, D), lambda i: (0, 0)),
            pl.BlockSpec((BE, 8 * D_EDGE), lambda i: (i, 0)),
            pl.BlockSpec((8 * D_EDGE, 8 * D), lambda i: (0, 0)),
        ],
        out_specs=[
            pl.BlockSpec((BN, D), lambda i: (i, 0)),
            pl.BlockSpec((BN, D), lambda i: (i, 0)),
            pl.BlockSpec((BE, 8 * D), lambda i: (i, 0)),
        ],
        out_shape=[
            jax.ShapeDtypeStruct((N, D), jnp.float32),
            jax.ShapeDtypeStruct((N, D), jnp.float32),
            jax.ShapeDtypeStruct((E // 8, 8 * D), jnp.float32),
        ],
    )(x, W_l, b_l.reshape(1, D), W_r, b_r.reshape(1, D), ea2, w_bd)
    return xl, xr, e.reshape(E, D)


def _tc_combine(U, denT, bias):
    BN = 1000
    return pl.pallas_call(
        _combine_body,
        grid=(N // BN,),
        in_specs=[
            pl.BlockSpec((1, BN, D), lambda i: (0, i, 0)),
            pl.BlockSpec((1, BN, D), lambda i: (1, i, 0)),
            pl.BlockSpec((BN, NW), lambda i: (i, 0)),
            pl.BlockSpec((1, D), lambda i: (0, 0)),
        ],
        out_specs=pl.BlockSpec((BN, D), lambda i: (i, 0)),
        out_shape=jax.ShapeDtypeStruct((N, D), jnp.float32),
    )(U, U, denT, bias.reshape(1, D))


def _sc_body(ei_hbm, xl_hbm, xr_hbm, e_hbm, att_hbm,
             u_out, den_out,
             srcv, dstv, xlb0, xlb1, xrb0, xrb1, eb0, eb1, pb, attv,
             den_local, u_sh,
             semi_s, semi_d, semg_xl, semg_xr, semg_e, sem_s):
    cid = lax.axis_index("c")
    sid = lax.axis_index("s")
    wid = sid * NC + cid

    xlb = (xlb0, xlb1)
    xrb = (xrb0, xrb1)
    eb = (eb0, eb1)

    pltpu.sync_copy(att_hbm, attv)
    zero16 = jnp.zeros((16,), jnp.float32)

    def _zero_row(r, _):
        for j in range(D // 16):
            xlb0[r, pl.ds(16 * j, 16)] = zero16
        return 0

    lax.fori_loop(0, C, _zero_row, 0)

    def _zero_den(r, _):
        den_local[pl.ds(r * 16, 16)] = zero16
        return 0

    lax.fori_loop(0, N // 16, _zero_den, 0)

    for k in range(-(-NBLK // NS)):
        b = sid + NS * k

        @pl.when(b < NBLK)
        def _():
            pltpu.sync_copy(xlb0, u_sh.at[pl.ds(b * C, C)])

    plsc.subcore_barrier()

    att_regs = [attv[pl.ds(16 * j, 16)] for j in range(D // 16)]
    iota16 = lax.iota(jnp.int32, 16)

    def chunk_base(i):
        return (wid + NW * i) * C

    def issue_idx(i, islot):
        base = chunk_base(i)
        pltpu.async_copy(ei_hbm.at[pl.ds(base, C)], srcv.at[islot], semi_s[islot])
        pltpu.async_copy(ei_hbm.at[pl.ds(E + base, C)], dstv.at[islot], semi_d[islot])

    def wait_idx(islot):
        pltpu.make_async_copy(ei_hbm.at[pl.ds(0, C)], srcv.at[islot], semi_s[islot]).wait()
        pltpu.make_async_copy(ei_hbm.at[pl.ds(0, C)], dstv.at[islot], semi_d[islot]).wait()

    def issue_gathers(i, islot, gslot):
        base = chunk_base(i)
        pltpu.async_copy(xl_hbm.at[srcv.at[islot]], xlb[gslot], semg_xl[gslot])
        pltpu.async_copy(xr_hbm.at[dstv.at[islot]], xrb[gslot], semg_xr[gslot])
        pltpu.async_copy(e_hbm.at[pl.ds(base, C)], eb[gslot], semg_e[gslot])

    def wait_gathers(islot, gslot):
        pltpu.make_async_copy(xl_hbm.at[srcv.at[islot]], xlb[gslot], semg_xl[gslot]).wait()
        pltpu.make_async_copy(xr_hbm.at[dstv.at[islot]], xrb[gslot], semg_xr[gslot]).wait()
        pltpu.make_async_copy(e_hbm.at[pl.ds(0, C)], eb[gslot], semg_e[gslot]).wait()

    def issue_scatter(islot, gslot):
        pltpu.async_copy(xlb[gslot], u_sh.at[dstv.at[islot]], sem_s[gslot], add=True)

    def wait_scatter(islot, gslot):
        pltpu.make_async_copy(xlb[gslot], u_sh.at[dstv.at[islot]], sem_s[gslot]).wait()

    def compute(islot, gslot):
        xlc, xrc, ec = xlb[gslot], xrb[gslot], eb[gslot]

        @plsc.parallel_loop(0, C, unroll=4)
        def _edge_a(ci):
            acc = None
            for j in range(D // 16):
                sl = pl.ds(16 * j, 16)
                z = xlc[ci, sl] + xrc[ci, sl] + ec[ci, sl]
                z = jnp.maximum(z, NEG * z)
                acc = att_regs[j] * z if acc is None else acc + att_regs[j] * z
            base = ci * 16
            pb[pl.ds(base, 16)] = acc
            s1 = acc + pb[pl.ds(base + 8, 16)]
            pb[pl.ds(base, 16)] = s1
            s2 = s1 + pb[pl.ds(base + 4, 16)]
            pb[pl.ds(base, 16)] = s2
            s3 = s2 + pb[pl.ds(base + 2, 16)]
            pb[pl.ds(base, 16)] = s3
            s4 = s3 + pb[pl.ds(base + 1, 16)]
            pb[pl.ds(base, 16)] = jnp.full((16,), s4[0], jnp.float32)

        @plsc.parallel_loop(0, C, unroll=4)
        def _edge_c(ci):
            pc = jnp.exp(pb[pl.ds(ci * 16, 16)])
            pb[pl.ds(ci * 16, 16)] = pc
            for j in range(D // 16):
                sl = pl.ds(16 * j, 16)
                xlc[ci, sl] = pc * xlc[ci, sl]

        for g, (doff, lanes) in enumerate([(0, range(16)), (16, range(16)),
                                           (24, range(8, 16))]):
            dvec = dstv[islot, pl.ds(doff, 16)]
            for l in lanes:
                ci = doff + l
                pc = pb[pl.ds(ci * 16, 16)]
                d = dvec[l]
                dbase = (d // 16) * 16
                loff = d - dbase
                sl = pl.ds(dbase, 16)
                den_local[sl] = den_local[sl] + jnp.where(
                    iota16 == loff, 1.0, 0.0) * pc

    issue_idx(0, 0)
    issue_idx(1, 1)
    wait_idx(0)
    issue_gathers(0, 0, 0)

    def _quad(k4, _):
        for q in range(4):
            i = 4 * k4 + q
            islot = q
            gslot = q % 2

            @pl.when(i < NPW)
            def _():
                wait_gathers(islot, gslot)

                @pl.when(i + 1 < NPW)
                def _():
                    @pl.when(i >= 1)
                    def _():
                        wait_scatter((islot + 3) % 4, (gslot + 1) % 2)
                    wait_idx((islot + 1) % 4)
                    issue_gathers(i + 1, (islot + 1) % 4, (gslot + 1) % 2)

                @pl.when(i + 2 < NPW)
                def _():
                    issue_idx(i + 2, (islot + 2) % 4)

                compute(islot, gslot)
                issue_scatter(islot, gslot)

        return 0

    lax.fori_loop(0, (NPW + 3) // 4, _quad, 0)

    wait_scatter((NPW - 1) % 4, (NPW - 1) % 2)

    pltpu.sync_copy(den_local, den_out.at[wid])

    plsc.subcore_barrier()

    for k in range(-(-NBLK // NS)):
        b = sid + NS * k

        @pl.when(b < NBLK)
        def _():
            pltpu.sync_copy(u_sh.at[pl.ds(b * C, C)], u_out.at[cid, pl.ds(b * C, C)])


def _sc_edge_phase(ei, xl, xr, e, attv):
    mesh = plsc.VectorSubcoreMesh(core_axis_name="c", subcore_axis_name="s",
                                  num_cores=NC, num_subcores=NS)
    f = pl.kernel(
        _sc_body,
        out_type=[
            jax.ShapeDtypeStruct((NC, N, D), jnp.float32),
            jax.ShapeDtypeStruct((NW, N), jnp.float32),
        ],
        mesh=mesh,
        scratch_types=[
            pltpu.VMEM((4, C), jnp.int32),
            pltpu.VMEM((4, C), jnp.int32),
            pltpu.VMEM((C, D), jnp.float32),
            pltpu.VMEM((C, D), jnp.float32),
            pltpu.VMEM((C, D), jnp.float32),
            pltpu.VMEM((C, D), jnp.float32),
            pltpu.VMEM((C, D), jnp.float32),
            pltpu.VMEM((C, D), jnp.float32),
            pltpu.VMEM((C * 16 + 16,), jnp.float32),
            pltpu.VMEM((D,), jnp.float32),
            pltpu.VMEM((N,), jnp.float32),
            pltpu.VMEM_SHARED((N, D), jnp.float32),
            [pltpu.SemaphoreType.DMA] * 4,
            [pltpu.SemaphoreType.DMA] * 4,
            [pltpu.SemaphoreType.DMA] * 2,
            [pltpu.SemaphoreType.DMA] * 2,
            [pltpu.SemaphoreType.DMA] * 2,
            [pltpu.SemaphoreType.DMA] * 2,
        ],
    )
    return f(ei, xl, xr, e, attv)


def kernel(x, edge_index, edge_attr, W_l, b_l, W_r, b_r, W_e, att, bias):
    xl, xr, e = _tc_matmuls(x, W_l, b_l, W_r, b_r, edge_attr, W_e)
    U, den = _sc_edge_phase(edge_index.reshape(2 * E), xl, xr, e, att.reshape(D))
    return _tc_combine(U, den.T, bias)


# speedup vs baseline: 1.2003x; 1.2003x over previous
import jax
import jax.numpy as jnp
from jax import lax
from jax.experimental import pallas as pl
from jax.experimental.pallas import tpu as pltpu
from jax.experimental.pallas import tpu_sc as plsc

N = 10000
E = 160000
D = 128
DA = 144
D_EDGE = 16
NEG = 0.2

NC = 2
NS = 16
NW = NC * NS
C = 40
NCHUNK = E // C
NPW = NCHUNK // NW
NBLK = N // C


def _fused_mm_body(x_ref, wl_ref, bl_ref, wr_ref, br_ref, ea_ref, we_ref,
                   xl_ref, xr_ref, e_ref):
    x = x_ref[...]
    xl_ref[...] = jnp.dot(x, wl_ref[...], preferred_element_type=jnp.float32) + bl_ref[...]
    xr_ref[...] = jnp.dot(x, wr_ref[...], preferred_element_type=jnp.float32) + br_ref[...]
    e_ref[...] = jnp.dot(ea_ref[...], we_ref[...], preferred_element_type=jnp.float32)


def _combine_body(u0_ref, u1_ref, dt_ref, b_ref, o_ref):
    den = jnp.sum(dt_ref[...], axis=1, keepdims=True)
    o_ref[...] = (u0_ref[0] + u1_ref[0]) / (den + 1e-16) + b_ref[...]


def _tc_matmuls(x, W_l, b_l, W_r, b_r, edge_attr, W_e):
    BN = 1000
    BE = 16000
    G = 10
    xl, xr, e = pl.pallas_call(
        _fused_mm_body,
        grid=(G,),
        in_specs=[
            pl.BlockSpec((BN, D), lambda i: (i, 0)),
            pl.BlockSpec((D, D), lambda i: (0, 0)),
            pl.BlockSpec((1, D), lambda i: (0, 0)),
            pl.BlockSpec((D, D), lambda i: (0, 0)),
            pl.BlockSpec((1, D), lambda i: (0, 0)),
            pl.BlockSpec((BE, D_EDGE), lambda i: (i, 0)),
            pl.BlockSpec((D_EDGE, D), lambda i: (0, 0)),
        ],
        out_specs=[
            pl.BlockSpec((BN, D), lambda i: (i, 0)),
            pl.BlockSpec((BN, D), lambda i: (i, 0)),
            pl.BlockSpec((BE, D), lambda i: (i, 0)),
        ],
        out_shape=[
            jax.ShapeDtypeStruct((N, D), jnp.float32),
            jax.ShapeDtypeStruct((N, D), jnp.float32),
            jax.ShapeDtypeStruct((E, D), jnp.float32),
        ],
    )(x, W_l, b_l.reshape(1, D), W_r, b_r.reshape(1, D), edge_attr, W_e)
    return xl, xr, e


def _tc_combine(U, denT, bias):
    BN = 1000
    return pl.pallas_call(
        _combine_body,
        grid=(N // BN,),
        in_specs=[
            pl.BlockSpec((1, BN, D), lambda i: (0, i, 0)),
            pl.BlockSpec((1, BN, D), lambda i: (1, i, 0)),
            pl.BlockSpec((BN, NW), lambda i: (i, 0)),
            pl.BlockSpec((1, D), lambda i: (0, 0)),
        ],
        out_specs=pl.BlockSpec((BN, D), lambda i: (i, 0)),
        out_shape=jax.ShapeDtypeStruct((N, D), jnp.float32),
    )(U, U, denT, bias.reshape(1, D))


def _sc_body(ei_hbm, xl_hbm, xr_hbm, e_hbm, att_hbm,
             u_out, den_out,
             srcv, dstv, xlb0, xlb1, xrb0, xrb1, eb0, eb1, pb, attv,
             den_local, u_sh,
             semi_s, semi_d, semg_xl, semg_xr, semg_e, sem_s):
    cid = lax.axis_index("c")
    sid = lax.axis_index("s")
    wid = sid * NC + cid

    xlb = (xlb0, xlb1)
    xrb = (xrb0, xrb1)
    eb = (eb0, eb1)

    pltpu.sync_copy(att_hbm, attv)
    zero16 = jnp.zeros((16,), jnp.float32)

    def _zero_row(r, _):
        for j in range(D // 16):
            xlb0[r, pl.ds(16 * j, 16)] = zero16
        return 0

    lax.fori_loop(0, C, _zero_row, 0)

    def _zero_den(r, _):
        den_local[pl.ds(r * 16, 16)] = zero16
        return 0

    lax.fori_loop(0, N // 16, _zero_den, 0)

    for k in range(-(-NBLK // NS)):
        b = sid + NS * k

        @pl.when(b < NBLK)
        def _():
            pltpu.sync_copy(xlb0, u_sh.at[pl.ds(b * C, C)])

    plsc.subcore_barrier()

    att_regs = [attv[pl.ds(16 * j, 16)] for j in range(D // 16)]
    iota16 = lax.iota(jnp.int32, 16)

    def chunk_base(i):
        return (wid + NW * i) * C

    def issue_idx(i, islot):
        base = chunk_base(i)
        pltpu.async_copy(ei_hbm.at[pl.ds(base, C)], srcv.at[islot], semi_s[islot])
        pltpu.async_copy(ei_hbm.at[pl.ds(E + base, C)], dstv.at[islot], semi_d[islot])

    def wait_idx(islot):
        pltpu.make_async_copy(ei_hbm.at[pl.ds(0, C)], srcv.at[islot], semi_s[islot]).wait()
        pltpu.make_async_copy(ei_hbm.at[pl.ds(0, C)], dstv.at[islot], semi_d[islot]).wait()

    def issue_gathers(i, islot, gslot):
        base = chunk_base(i)
        pltpu.async_copy(xl_hbm.at[srcv.at[islot]], xlb[gslot], semg_xl[gslot])
        pltpu.async_copy(xr_hbm.at[dstv.at[islot]], xrb[gslot], semg_xr[gslot])
        pltpu.async_copy(e_hbm.at[pl.ds(base, C)], eb[gslot], semg_e[gslot])

    def wait_gathers(islot, gslot):
        pltpu.make_async_copy(xl_hbm.at[srcv.at[islot]], xlb[gslot], semg_xl[gslot]).wait()
        pltpu.make_async_copy(xr_hbm.at[dstv.at[islot]], xrb[gslot], semg_xr[gslot]).wait()
        pltpu.make_async_copy(e_hbm.at[pl.ds(0, C)], eb[gslot], semg_e[gslot]).wait()

    def issue_scatter(islot, gslot):
        pltpu.async_copy(xlb[gslot], u_sh.at[dstv.at[islot]], sem_s[gslot], add=True)

    def wait_scatter(islot, gslot):
        pltpu.make_async_copy(xlb[gslot], u_sh.at[dstv.at[islot]], sem_s[gslot]).wait()

    def compute(islot, gslot):
        xlc, xrc, ec = xlb[gslot], xrb[gslot], eb[gslot]

        @plsc.parallel_loop(0, C, unroll=4)
        def _edge_a(ci):
            acc = None
            for j in range(D // 16):
                sl = pl.ds(16 * j, 16)
                z = xlc[ci, sl] + xrc[ci, sl] + ec[ci, sl]
                z = jnp.maximum(z, NEG * z)
                acc = att_regs[j] * z if acc is None else acc + att_regs[j] * z
            base = ci * 16
            pb[pl.ds(base, 16)] = acc
            s1 = acc + pb[pl.ds(base + 8, 16)]
            pb[pl.ds(base, 16)] = s1
            s2 = s1 + pb[pl.ds(base + 4, 16)]
            pb[pl.ds(base, 16)] = s2
            s3 = s2 + pb[pl.ds(base + 2, 16)]
            pb[pl.ds(base, 16)] = s3
            s4 = s3 + pb[pl.ds(base + 1, 16)]
            pb[pl.ds(base, 16)] = jnp.full((16,), s4[0], jnp.float32)

        @plsc.parallel_loop(0, C, unroll=4)
        def _edge_c(ci):
            pc = jnp.exp(pb[pl.ds(ci * 16, 16)])
            pb[pl.ds(ci * 16, 16)] = pc
            for j in range(D // 16):
                sl = pl.ds(16 * j, 16)
                xlc[ci, sl] = pc * xlc[ci, sl]

        for g, (doff, lanes) in enumerate([(0, range(16)), (16, range(16)),
                                           (24, range(8, 16))]):
            dvec = dstv[islot, pl.ds(doff, 16)]
            for l in lanes:
                ci = doff + l
                pc = pb[pl.ds(ci * 16, 16)]
                d = dvec[l]
                dbase = (d // 16) * 16
                loff = d - dbase
                sl = pl.ds(dbase, 16)
                den_local[sl] = den_local[sl] + jnp.where(
                    iota16 == loff, 1.0, 0.0) * pc

    issue_idx(0, 0)
    issue_idx(1, 1)
    wait_idx(0)
    issue_gathers(0, 0, 0)

    def _quad(k4, _):
        for q in range(4):
            i = 4 * k4 + q
            islot = q
            gslot = q % 2

            @pl.when(i < NPW)
            def _():
                wait_gathers(islot, gslot)

                @pl.when(i + 1 < NPW)
                def _():
                    @pl.when(i >= 1)
                    def _():
                        wait_scatter((islot + 3) % 4, (gslot + 1) % 2)
                    wait_idx((islot + 1) % 4)
                    issue_gathers(i + 1, (islot + 1) % 4, (gslot + 1) % 2)

                @pl.when(i + 2 < NPW)
                def _():
                    issue_idx(i + 2, (islot + 2) % 4)

                compute(islot, gslot)
                issue_scatter(islot, gslot)

        return 0

    lax.fori_loop(0, (NPW + 3) // 4, _quad, 0)

    wait_scatter((NPW - 1) % 4, (NPW - 1) % 2)

    pltpu.sync_copy(den_local, den_out.at[wid])

    plsc.subcore_barrier()

    for k in range(-(-NBLK // NS)):
        b = sid + NS * k

        @pl.when(b < NBLK)
        def _():
            pltpu.sync_copy(u_sh.at[pl.ds(b * C, C)], u_out.at[cid, pl.ds(b * C, C)])


def _sc_edge_phase(ei, xl, xr, e, attv):
    mesh = plsc.VectorSubcoreMesh(core_axis_name="c", subcore_axis_name="s",
                                  num_cores=NC, num_subcores=NS)
    f = pl.kernel(
        _sc_body,
        out_type=[
            jax.ShapeDtypeStruct((NC, N, D), jnp.float32),
            jax.ShapeDtypeStruct((NW, N), jnp.float32),
        ],
        mesh=mesh,
        scratch_types=[
            pltpu.VMEM((4, C), jnp.int32),
            pltpu.VMEM((4, C), jnp.int32),
            pltpu.VMEM((C, D), jnp.float32),
            pltpu.VMEM((C, D), jnp.float32),
            pltpu.VMEM((C, D), jnp.float32),
            pltpu.VMEM((C, D), jnp.float32),
            pltpu.VMEM((C, D), jnp.float32),
            pltpu.VMEM((C, D), jnp.float32),
            pltpu.VMEM((C * 16 + 16,), jnp.float32),
            pltpu.VMEM((D,), jnp.float32),
            pltpu.VMEM((N,), jnp.float32),
            pltpu.VMEM_SHARED((N, D), jnp.float32),
            [pltpu.SemaphoreType.DMA] * 4,
            [pltpu.SemaphoreType.DMA] * 4,
            [pltpu.SemaphoreType.DMA] * 2,
            [pltpu.SemaphoreType.DMA] * 2,
            [pltpu.SemaphoreType.DMA] * 2,
            [pltpu.SemaphoreType.DMA] * 2,
        ],
    )
    return f(ei, xl, xr, e, attv)


def kernel(x, edge_index, edge_attr, W_l, b_l, W_r, b_r, W_e, att, bias):
    xl, xr, e = _tc_matmuls(x, W_l, b_l, W_r, b_r, edge_attr, W_e)
    U, den = _sc_edge_phase(edge_index.reshape(2 * E), xl, xr, e, att.reshape(D))
    return _tc_combine(U, den.T, bias)
